# Initial kernel scaffold; baseline (speedup 1.0000x reference)
import jax
import jax.numpy as jnp
from jax.experimental import pallas as pl


def kernel(x, g_w, g_b, theta_w, theta_b, phi_w, phi_b, match_w, match_b, restore_w, restore_b):
    raise NotImplementedError("write your pallas kernel here")


# speedup vs baseline: 2.4695x; 2.4695x over previous
import functools

import jax
import jax.numpy as jnp
import numpy as np
from jax.experimental import pallas as pl

KSIZE = 7
STRIDE = 4
SCALE = 10.0
TOPM = 512
L = 3025
LP = 3072
D = 784
ROWT = 128


def _attn_kernel(p1_ref, p2_ref, p3_ref, out_ref):
    p1t = p1_ref[0]
    p2b = p2_ref[0]
    p3b = p3_ref[0]

    score = jax.lax.dot_general(
        p1t, p2b, (((1,), (1,)), ((), ())),
        preferred_element_type=jnp.float32)

    col = jax.lax.broadcasted_iota(jnp.int32, (ROWT, LP), 1)
    real = col < L

    mneg = jnp.where(real, score, -jnp.inf)
    mx = jnp.maximum(jnp.max(mneg, axis=1, keepdims=True) * SCALE, 0.0)

    u = jax.lax.bitcast_convert_type(score, jnp.uint32)
    key = jnp.where(u >= jnp.uint32(0x80000000), ~u,
                    u | jnp.uint32(0x80000000))
    key = jnp.where(real, key, jnp.uint32(0))

    def step(i, acc):
        bit = jax.lax.shift_left(jnp.uint32(1), jnp.uint32(31) - i.astype(jnp.uint32))
        cand = acc | bit
        cnt = jnp.sum((key >= cand).astype(jnp.int32), axis=1, keepdims=True)
        return jnp.where(cnt >= TOPM, cand, acc)

    acc = jax.lax.fori_loop(0, 32, step, jnp.zeros((ROWT, 1), jnp.uint32))

    sel = key >= acc
    num = jnp.where(sel, jnp.exp(score * SCALE - mx), jnp.exp(-mx))
    num = jnp.where(real, num, 0.0)
    z = jnp.sum(num, axis=1, keepdims=True)
    attn = num / z

    out_ref[0] = jax.lax.dot_general(
        attn, p3b, (((1,), (0,)), ((), ())),
        preferred_element_type=jnp.float32)


def _fused_attention(p1, p2, p3):
    B = p1.shape[0]
    pad = [(0, 0), (0, LP - L), (0, 0)]
    p1 = jnp.pad(p1, pad)
    p2 = jnp.pad(p2, pad)
    p3 = jnp.pad(p3, pad)
    grid = (B, LP // ROWT)
    agg = pl.pallas_call(
        _attn_kernel,
        grid=grid,
        in_specs=[
            pl.BlockSpec((1, ROWT, D), lambda b, t: (b, t, 0)),
            pl.BlockSpec((1, LP, D), lambda b, t: (b, 0, 0)),
            pl.BlockSpec((1, LP, D), lambda b, t: (b, 0, 0)),
        ],
        out_specs=pl.BlockSpec((1, ROWT, D), lambda b, t: (b, t, 0)),
        out_shape=jax.ShapeDtypeStruct((B, LP, D), jnp.float32),
    )(p1, p2, p3)
    return agg[:, :L, :]


def _conv2d(x, w, b, pad):
    out = jax.lax.conv_general_dilated(
        x, w, window_strides=(1, 1), padding=[(pad, pad), (pad, pad)],
        dimension_numbers=('NCHW', 'OIHW', 'NCHW'))
    return out + b[None, :, None, None]


def _unfold(x, k, s):
    B, C, H, W = x.shape
    Ho = (H - k) // s + 1
    Wo = (W - k) // s + 1
    ri = (jnp.arange(Ho) * s)[:, None, None, None] + jnp.arange(k)[None, None, :, None]
    ci = (jnp.arange(Wo) * s)[None, :, None, None] + jnp.arange(k)[None, None, None, :]
    p = x[:, :, ri, ci]
    p = p.transpose(0, 1, 4, 5, 2, 3).reshape(B, C * k * k, Ho * Wo)
    return p


def _fold(p, H, W, k, s):
    B, CKK, Lx = p.shape
    C = CKK // (k * k)
    Ho = (H - k) // s + 1
    Wo = (W - k) // s + 1
    pr = p.reshape(B, C, k, k, Ho, Wo).transpose(0, 1, 4, 5, 2, 3)
    ri = (jnp.arange(Ho) * s)[:, None, None, None] + jnp.arange(k)[None, None, :, None]
    ci = (jnp.arange(Wo) * s)[None, :, None, None] + jnp.arange(k)[None, None, None, :]
    out = jnp.zeros((B, C, H, W), dtype=p.dtype)
    out = out.at[:, :, ri, ci].add(pr)
    return out


@jax.jit
def kernel(x, g_w, g_b, theta_w, theta_b, phi_w, phi_b, match_w, match_b,
           restore_w, restore_b):
    B, C, H, W = x.shape
    b1 = _conv2d(x, g_w, g_b, 1)
    b2 = _conv2d(x, theta_w, theta_b, 0)
    b3 = _conv2d(x, phi_w, phi_b, 0)
    p1 = _unfold(b1, KSIZE, STRIDE).transpose(0, 2, 1)
    p2 = _unfold(b2, KSIZE, STRIDE).transpose(0, 2, 1)
    p3 = _unfold(b3, KSIZE, STRIDE).transpose(0, 2, 1)

    agg = _fused_attention(p1, p2, p3).transpose(0, 2, 1)

    out = _fold(agg, H, W, KSIZE, STRIDE)

    hh = np.arange(H)
    cov = np.zeros(H, np.float32)
    for i in range(0, (H - KSIZE) // STRIDE + 1):
        cov[i * STRIDE:i * STRIDE + KSIZE] += 1.0
    cnt = jnp.asarray(cov)
    cnt2d = cnt[:, None] * cnt[None, :]
    mw = match_w[:, :, 0, 0].sum(axis=1)
    mask = cnt2d[None, None] * mw[None, :, None, None] + match_b[None, :, None, None]
    out = out / (mask + 1e-08)
    out = jnp.einsum('oc,bchw->bohw', restore_w[:, :, 0, 0], out) + restore_b[None, :, None, None]
    return out


# speedup vs baseline: 13.5808x; 5.4994x over previous
import functools

import jax
import jax.numpy as jnp
import numpy as np
from jax.experimental import pallas as pl

KSIZE = 7
STRIDE = 4
SCALE = 10.0
TOPM = 512
PH = 55
L = PH * PH
D = 784
CI = 16
ROWT = 128


def _conv_unfold_kernel(xa_ref, xb_ref, xc_ref, gw_ref, gb_ref, tw_ref,
                        tb_ref, pw_ref, pb_ref, p1_ref, p2_ref, p3_ref):
    x12 = jnp.concatenate([xa_ref[0, :, 0], xb_ref[0, :, 0], xc_ref[0, :, 0]],
                          axis=1)

    acc = None
    for k in range(9):
        ky, kx = divmod(k, 3)
        w = gw_ref[k]
        xs = x12[:, ky:ky + 7, kx:kx + 224]
        t = jax.lax.dot_general(w, xs, (((1,), (0,)), ((), ())),
                                preferred_element_type=jnp.float32)
        acc = t if acc is None else acc + t
    b1 = acc + gb_ref[...].reshape(CI, 1, 1)

    x7 = x12[:, 1:8, 1:225]
    b2 = jax.lax.dot_general(tw_ref[...], x7, (((1,), (0,)), ((), ())),
                             preferred_element_type=jnp.float32)
    b2 = b2 + tb_ref[...].reshape(CI, 1, 1)
    b3 = jax.lax.dot_general(pw_ref[...], x7, (((1,), (0,)), ((), ())),
                             preferred_element_type=jnp.float32)
    b3 = b3 + pb_ref[...].reshape(CI, 1, 1)

    def to_patches(bm):
        br = bm.reshape(CI, KSIZE, PH + 1, STRIDE)
        parts = []
        for ky in range(KSIZE):
            for kx in range(KSIZE):
                q, r = divmod(kx, STRIDE)
                parts.append(br[:, ky, q:q + PH, r])
        return jnp.concatenate(parts, axis=0)

    p1_ref[0, 0] = to_patches(b1)
    p2_ref[0, 0] = to_patches(b2)
    p3_ref[0, 0] = to_patches(b3)


def _conv_unfold(x, g_w, g_b, theta_w, theta_b, phi_w, phi_b):
    B = x.shape[0]
    xp = jnp.pad(x, ((0, 0), (0, 0), (1, 3), (1, 1)))
    xp = xp.reshape(B, 96, 57, 4, 226)
    gw = g_w.transpose(2, 3, 0, 1).reshape(9, CI, 96)
    tw = theta_w.reshape(CI, 96)
    pw = phi_w.reshape(CI, 96)
    gb = g_b.reshape(CI, 1)
    tb = theta_b.reshape(CI, 1)
    pb = phi_b.reshape(CI, 1)

    def xspec(off):
        return pl.BlockSpec((1, 96, 1, 4, 226),
                            lambda b, i: (b, 0, i + off, 0, 0))
    wspec = lambda s: pl.BlockSpec(s, lambda b, i: tuple(0 for _ in s))
    pspec = pl.BlockSpec((1, 1, D, PH), lambda b, i: (b, i, 0, 0))
    pshape = jax.ShapeDtypeStruct((B, PH, D, PH), jnp.float32)

    slabs = pl.pallas_call(
        _conv_unfold_kernel,
        grid=(B, PH),
        in_specs=[xspec(0), xspec(1), xspec(2),
                  wspec((9, CI, 96)), wspec((CI, 1)),
                  wspec((CI, 96)), wspec((CI, 1)),
                  wspec((CI, 96)), wspec((CI, 1))],
        out_specs=[pspec, pspec, pspec],
        out_shape=[pshape, pshape, pshape],
    )(xp, xp, xp, gw, gb, tw, tb, pw, pb)
    return [s.transpose(0, 2, 1, 3).reshape(B, D, L) for s in slabs]


def _attn_kernel(p1_ref, p2_ref, p3_ref, out_ref):
    p1t = p1_ref[0]
    p2b = p2_ref[0]
    p3b = p3_ref[0]

    score = jax.lax.dot_general(
        p1t, p2b, (((0,), (0,)), ((), ())),
        preferred_element_type=jnp.float32)

    mx = jnp.maximum(jnp.max(score, axis=1, keepdims=True) * SCALE, 0.0)

    u = jax.lax.bitcast_convert_type(score, jnp.uint32)
    key = jnp.where(u >= jnp.uint32(0x80000000), ~u,
                    u | jnp.uint32(0x80000000))

    def step(i, acc):
        bit = jax.lax.shift_left(jnp.uint32(1),
                                 jnp.uint32(31) - i.astype(jnp.uint32))
        cand = acc | bit
        cnt = jnp.sum((key >= cand).astype(jnp.int32), axis=1, keepdims=True)
        return jnp.where(cnt >= TOPM, cand, acc)

    acc = jax.lax.fori_loop(0, 32, step,
                            jnp.zeros((p1t.shape[1], 1), jnp.uint32))

    sel = key >= acc
    num = jnp.where(sel, jnp.exp(score * SCALE - mx), jnp.exp(-mx))
    z = jnp.sum(num, axis=1, keepdims=True)
    attn = num / z

    out_ref[0] = jax.lax.dot_general(
        p3b, attn, (((1,), (1,)), ((), ())),
        preferred_element_type=jnp.float32)


def _fused_attention(p1t, p2t, p3t):
    B = p1t.shape[0]
    grid = (B, pl.cdiv(L, ROWT))
    return pl.pallas_call(
        _attn_kernel,
        grid=grid,
        in_specs=[
            pl.BlockSpec((1, D, ROWT), lambda b, t: (b, 0, t)),
            pl.BlockSpec((1, D, L), lambda b, t: (b, 0, 0)),
            pl.BlockSpec((1, D, L), lambda b, t: (b, 0, 0)),
        ],
        out_specs=pl.BlockSpec((1, D, ROWT), lambda b, t: (b, 0, t)),
        out_shape=jax.ShapeDtypeStruct((B, D, L), jnp.float32),
    )(p1t, p2t, p3t)


def _fold_kernel(acur_ref, aprev_ref, cntc_ref, mw_ref, mb_ref,
                 rw_ref, rb_ref, out_ref):
    g = pl.program_id(1)
    acur = acur_ref[0, 0]
    aprev = aprev_ref[0, 0]
    wc = jnp.where(g <= PH - 1, 1.0, 0.0).astype(jnp.float32)
    wp = jnp.where(g >= 1, 1.0, 0.0).astype(jnp.float32)

    buf = [[jnp.zeros((CI, PH + 1), jnp.float32) for _ in range(STRIDE)]
           for _ in range(4)]
    for src, wgt, dyoff, tmax in ((acur, wc, 0, 4), (aprev, wp, 4, 3)):
        for t in range(tmax):
            dy = t + dyoff
            for kx in range(KSIZE):
                q, r = divmod(kx, STRIDE)
                f0 = (dy * KSIZE + kx) * CI
                s = src[f0:f0 + CI, :] * wgt
                buf[t][r] = buf[t][r] + jnp.pad(s, ((0, 0), (q, 1 - q)))

    rows = [jnp.stack(buf[t], axis=-1).reshape(CI, 224) for t in range(4)]
    folded = jnp.stack(rows, axis=1)

    gmin = jnp.minimum(g, PH - 1)
    c_low = (gmin - jnp.maximum(g - 1, 0) + 1).astype(jnp.float32)
    c_3 = (gmin - g + 1).astype(jnp.float32)
    cntr = jnp.stack([c_low, c_low, c_low, c_3]).reshape(1, 4, 1)

    denom = (cntr * cntc_ref[...].reshape(1, 1, 224)
             * mw_ref[...].reshape(CI, 1, 1)
             + mb_ref[...].reshape(CI, 1, 1) + 1e-08)
    normed = folded / denom

    out = jax.lax.dot_general(rw_ref[...], normed, (((1,), (0,)), ((), ())),
                              preferred_element_type=jnp.float32)
    out_ref[0, :, 0] = out + rb_ref[...].reshape(96, 1, 1)


def _fold_mask_restore(aggt, match_w, match_b, restore_w, restore_b):
    B = aggt.shape[0]
    a4 = aggt.reshape(B, D, PH, PH).transpose(0, 2, 1, 3)
    cov = np.zeros(224, np.float32)
    for i in range(PH):
        cov[i * STRIDE:i * STRIDE + KSIZE] += 1.0
    cntc = jnp.asarray(cov).reshape(1, 224)
    mw = match_w[:, :, 0, 0].sum(axis=1).reshape(CI, 1)
    mb = match_b.reshape(CI, 1)
    rw = restore_w.reshape(96, CI)
    rb = restore_b.reshape(96, 1)

    wspec = lambda s: pl.BlockSpec(s, lambda b, g: tuple(0 for _ in s))
    out = pl.pallas_call(
        _fold_kernel,
        grid=(B, 56),
        in_specs=[
            pl.BlockSpec((1, 1, D, PH),
                         lambda b, g: (b, jnp.minimum(g, PH - 1), 0, 0)),
            pl.BlockSpec((1, 1, D, PH),
                         lambda b, g: (b, jnp.maximum(g - 1, 0), 0, 0)),
            wspec((1, 224)), wspec((CI, 1)), wspec((CI, 1)),
            wspec((96, CI)), wspec((96, 1)),
        ],
        out_specs=pl.BlockSpec((1, 96, 1, 4, 224),
                               lambda b, g: (b, 0, g, 0, 0)),
        out_shape=jax.ShapeDtypeStruct((B, 96, 56, 4, 224), jnp.float32),
    )(a4, a4, cntc, mw, mb, rw, rb)
    return out.reshape(B, 96, 224, 224)


@jax.jit
def kernel(x, g_w, g_b, theta_w, theta_b, phi_w, phi_b, match_w, match_b,
           restore_w, restore_b):
    p1t, p2t, p3t = _conv_unfold(x, g_w, g_b, theta_w, theta_b,
                                 phi_w, phi_b)
    aggt = _fused_attention(p1t, p2t, p3t)
    return _fold_mask_restore(aggt, match_w, match_b, restore_w, restore_b)


# speedup vs baseline: 37.8241x; 2.7851x over previous
import functools

import jax
import jax.numpy as jnp
import numpy as np
from jax.experimental import pallas as pl

KSIZE = 7
STRIDE = 4
SCALE = 10.0
TOPM = 512
PH = 55
L = PH * PH
D = 784
CI = 16
ROWT = 256


def _conv_unfold_kernel(xa_ref, xb_ref, xc_ref, gw_ref, gb_ref, tw_ref,
                        tb_ref, pw_ref, pb_ref, sel_ref, p1_ref, p2_ref,
                        p3_ref):
    x12 = jnp.concatenate([xa_ref[0, :, 0], xb_ref[0, :, 0], xc_ref[0, :, 0]],
                          axis=1)

    acc = None
    for k in range(9):
        ky, kx = divmod(k, 3)
        w = gw_ref[k]
        xs = x12[:, ky:ky + 7, kx:kx + 224]
        t = jax.lax.dot_general(w, xs, (((1,), (0,)), ((), ())),
                                preferred_element_type=jnp.float32)
        acc = t if acc is None else acc + t
    b1 = acc + gb_ref[...].reshape(CI, 1, 1)

    x7 = x12[:, 1:8, 1:225]
    b2 = jax.lax.dot_general(tw_ref[...], x7, (((1,), (0,)), ((), ())),
                             preferred_element_type=jnp.float32)
    b2 = b2 + tb_ref[...].reshape(CI, 1, 1)
    b3 = jax.lax.dot_general(pw_ref[...], x7, (((1,), (0,)), ((), ())),
                             preferred_element_type=jnp.float32)
    b3 = b3 + pb_ref[...].reshape(CI, 1, 1)

    def to_patches(bm):
        parts = []
        for ky in range(KSIZE):
            r = jax.lax.dot_general(bm[:, ky, :], sel_ref[...],
                                    (((1,), (0,)), ((), ())),
                                    preferred_element_type=jnp.float32)
            for kx in range(KSIZE):
                parts.append(r[:, kx * 128:kx * 128 + PH])
        return jnp.concatenate(parts, axis=0)

    p1_ref[0, 0] = to_patches(b1)
    p2_ref[0, 0] = to_patches(b2)
    p3_ref[0, 0] = to_patches(b3)


def _conv_unfold(x, g_w, g_b, theta_w, theta_b, phi_w, phi_b):
    B = x.shape[0]
    xp = jnp.pad(x, ((0, 0), (0, 0), (1, 3), (1, 1)))
    xp = xp.reshape(B, 96, 57, 4, 226)
    gw = g_w.transpose(2, 3, 0, 1).reshape(9, CI, 96)
    tw = theta_w.reshape(CI, 96)
    pw = phi_w.reshape(CI, 96)
    gb = g_b.reshape(CI, 1)
    tb = theta_b.reshape(CI, 1)
    pb = phi_b.reshape(CI, 1)
    sel = np.zeros((224, 7 * 128), np.float32)
    for kx in range(KSIZE):
        for j in range(PH):
            sel[STRIDE * j + kx, kx * 128 + j] = 1.0
    sel = jnp.asarray(sel)

    def xspec(off):
        return pl.BlockSpec((1, 96, 1, 4, 226),
                            lambda b, i: (b, 0, i + off, 0, 0))
    wspec = lambda s: pl.BlockSpec(s, lambda b, i: tuple(0 for _ in s))
    pspec = pl.BlockSpec((1, 1, D, PH), lambda b, i: (b, i, 0, 0))
    pshape = jax.ShapeDtypeStruct((B, PH, D, PH), jnp.float32)

    slabs = pl.pallas_call(
        _conv_unfold_kernel,
        grid=(B, PH),
        in_specs=[xspec(0), xspec(1), xspec(2),
                  wspec((9, CI, 96)), wspec((CI, 1)),
                  wspec((CI, 96)), wspec((CI, 1)),
                  wspec((CI, 96)), wspec((CI, 1)),
                  wspec((224, 7 * 128))],
        out_specs=[pspec, pspec, pspec],
        out_shape=[pshape, pshape, pshape],
    )(xp, xp, xp, gw, gb, tw, tb, pw, pb, sel)
    return [s.transpose(0, 2, 1, 3).reshape(B, D, L) for s in slabs]


def _attn_kernel(p1_ref, p2_ref, p3_ref, out_ref):
    p1t = p1_ref[0]
    p2b = p2_ref[0]
    p3b = p3_ref[0]

    def split(a):
        hi = a.astype(jnp.bfloat16)
        lo = (a - hi.astype(jnp.float32)).astype(jnp.bfloat16)
        return hi, lo

    def dot3(a, b, dims):
        ah, al = split(a)
        bh, bl = split(b)
        out = jax.lax.dot_general(ah, bh, dims,
                                  preferred_element_type=jnp.float32)
        out += jax.lax.dot_general(ah, bl, dims,
                                   preferred_element_type=jnp.float32)
        out += jax.lax.dot_general(al, bh, dims,
                                   preferred_element_type=jnp.float32)
        return out

    score = dot3(p1t, p2b, (((0,), (0,)), ((), ())))

    mx = jnp.maximum(jnp.max(score, axis=1, keepdims=True) * SCALE, 0.0)

    u = jax.lax.bitcast_convert_type(score, jnp.uint32)
    key = jnp.where(u >= jnp.uint32(0x80000000), ~u,
                    u | jnp.uint32(0x80000000))

    def step(i, acc):
        bit = jax.lax.shift_left(jnp.uint32(1),
                                 jnp.uint32(31) - i.astype(jnp.uint32))
        cand = acc | bit
        cnt = jnp.sum((key >= cand).astype(jnp.int32), axis=1, keepdims=True)
        return jnp.where(cnt >= TOPM, cand, acc)

    acc = jax.lax.fori_loop(0, 32, step,
                            jnp.zeros((p1t.shape[1], 1), jnp.uint32))

    sel = key >= acc
    num = jnp.where(sel, jnp.exp(score * SCALE - mx), jnp.exp(-mx))
    z = jnp.sum(num, axis=1, keepdims=True)

    agg = dot3(p3b, num, (((1,), (1,)), ((), ())))
    out_ref[0] = agg * (1.0 / z).reshape(1, -1)


def _fused_attention(p1t, p2t, p3t):
    B = p1t.shape[0]
    grid = (B, pl.cdiv(L, ROWT))
    return pl.pallas_call(
        _attn_kernel,
        grid=grid,
        in_specs=[
            pl.BlockSpec((1, D, ROWT), lambda b, t: (b, 0, t)),
            pl.BlockSpec((1, D, L), lambda b, t: (b, 0, 0)),
            pl.BlockSpec((1, D, L), lambda b, t: (b, 0, 0)),
        ],
        out_specs=pl.BlockSpec((1, D, ROWT), lambda b, t: (b, 0, t)),
        out_shape=jax.ShapeDtypeStruct((B, D, L), jnp.float32),
    )(p1t, p2t, p3t)


def _fold_kernel(acur_ref, aprev_ref, vsel_ref, cntc_ref, mw_ref, mb_ref,
                 rw_ref, rb_ref, out_ref):
    g = pl.program_id(1)
    acur = acur_ref[0, 0]
    aprev = aprev_ref[0, 0]
    wc = jnp.where(g <= PH - 1, 1.0, 0.0).astype(jnp.float32)
    wp = jnp.where(g >= 1, 1.0, 0.0).astype(jnp.float32)

    def dy_row(src, dy):
        groups = [jnp.pad(src[(dy * KSIZE + kx) * CI:(dy * KSIZE + kx + 1) * CI, :],
                          ((0, 0), (0, 128 - PH))) for kx in range(KSIZE)]
        packed = jnp.concatenate(groups, axis=1)
        return jax.lax.dot_general(packed, vsel_ref[...],
                                   (((1,), (0,)), ((), ())),
                                   preferred_element_type=jnp.float32)

    rows = []
    for t in range(4):
        r = dy_row(acur, t) * wc
        if t < 3:
            r = r + dy_row(aprev, t + 4) * wp
        rows.append(r)
    folded = jnp.stack(rows, axis=1)

    gmin = jnp.minimum(g, PH - 1)
    c_low = (gmin - jnp.maximum(g - 1, 0) + 1).astype(jnp.float32)
    c_3 = (gmin - g + 1).astype(jnp.float32)
    cntr = jnp.stack([c_low, c_low, c_low, c_3]).reshape(1, 4, 1)

    denom = (cntr * cntc_ref[...].reshape(1, 1, 224)
             * mw_ref[...].reshape(CI, 1, 1)
             + mb_ref[...].reshape(CI, 1, 1) + 1e-08)
    normed = folded / denom

    out = jax.lax.dot_general(rw_ref[...], normed, (((1,), (0,)), ((), ())),
                              preferred_element_type=jnp.float32)
    out_ref[0, :, 0] = out + rb_ref[...].reshape(96, 1, 1)


def _fold_mask_restore(aggt, match_w, match_b, restore_w, restore_b):
    B = aggt.shape[0]
    a4 = aggt.reshape(B, D, PH, PH).transpose(0, 2, 1, 3)
    cov = np.zeros(224, np.float32)
    for i in range(PH):
        cov[i * STRIDE:i * STRIDE + KSIZE] += 1.0
    cntc = jnp.asarray(cov).reshape(1, 224)
    mw = match_w[:, :, 0, 0].sum(axis=1).reshape(CI, 1)
    mb = match_b.reshape(CI, 1)
    rw = restore_w.reshape(96, CI)
    rb = restore_b.reshape(96, 1)
    vsel = np.zeros((7 * 128, 224), np.float32)
    for kx in range(KSIZE):
        for j in range(PH):
            vsel[kx * 128 + j, STRIDE * j + kx] = 1.0
    vsel = jnp.asarray(vsel)

    wspec = lambda s: pl.BlockSpec(s, lambda b, g: tuple(0 for _ in s))
    out = pl.pallas_call(
        _fold_kernel,
        grid=(B, 56),
        in_specs=[
            pl.BlockSpec((1, 1, D, PH),
                         lambda b, g: (b, jnp.minimum(g, PH - 1), 0, 0)),
            pl.BlockSpec((1, 1, D, PH),
                         lambda b, g: (b, jnp.maximum(g - 1, 0), 0, 0)),
            wspec((7 * 128, 224)),
            wspec((1, 224)), wspec((CI, 1)), wspec((CI, 1)),
            wspec((96, CI)), wspec((96, 1)),
        ],
        out_specs=pl.BlockSpec((1, 96, 1, 4, 224),
                               lambda b, g: (b, 0, g, 0, 0)),
        out_shape=jax.ShapeDtypeStruct((B, 96, 56, 4, 224), jnp.float32),
    )(a4, a4, vsel, cntc, mw, mb, rw, rb)
    return out.reshape(B, 96, 224, 224)


@jax.jit
def kernel(x, g_w, g_b, theta_w, theta_b, phi_w, phi_b, match_w, match_b,
           restore_w, restore_b):
    p1t, p2t, p3t = _conv_unfold(x, g_w, g_b, theta_w, theta_b,
                                 phi_w, phi_b)
    aggt = _fused_attention(p1t, p2t, p3t)
    return _fold_mask_restore(aggt, match_w, match_b, restore_w, restore_b)


# speedup vs baseline: 41.7743x; 1.1044x over previous
import functools

import jax
import jax.numpy as jnp
import numpy as np
from jax.experimental import pallas as pl

KSIZE = 7
STRIDE = 4
SCALE = 10.0
TOPM = 512
PH = 55
L = PH * PH
D = 784
CI = 16
ROWT = 256


def _conv_unfold_kernel(xa_ref, xb_ref, xc_ref, gw_ref, gb_ref, tw_ref,
                        tb_ref, pw_ref, pb_ref, sel_ref, p1_ref, p2_ref,
                        p3_ref):
    x12 = jnp.concatenate([xa_ref[0, :, 0], xb_ref[0, :, 0], xc_ref[0, :, 0]],
                          axis=1)

    acc = None
    for k in range(9):
        ky, kx = divmod(k, 3)
        w = gw_ref[k]
        xs = x12[:, ky:ky + 7, kx:kx + 224]
        t = jax.lax.dot_general(w, xs, (((1,), (0,)), ((), ())),
                                preferred_element_type=jnp.float32)
        acc = t if acc is None else acc + t
    b1 = acc + gb_ref[...].reshape(CI, 1, 1)

    x7 = x12[:, 1:8, 1:225]
    b2 = jax.lax.dot_general(tw_ref[...], x7, (((1,), (0,)), ((), ())),
                             preferred_element_type=jnp.float32)
    b2 = b2 + tb_ref[...].reshape(CI, 1, 1)
    b3 = jax.lax.dot_general(pw_ref[...], x7, (((1,), (0,)), ((), ())),
                             preferred_element_type=jnp.float32)
    b3 = b3 + pb_ref[...].reshape(CI, 1, 1)

    def to_patches(bm):
        parts = []
        for ky in range(KSIZE):
            r = jax.lax.dot_general(bm[:, ky, :], sel_ref[...],
                                    (((1,), (0,)), ((), ())),
                                    preferred_element_type=jnp.float32)
            for kx in range(KSIZE):
                parts.append(r[:, kx * 128:kx * 128 + PH])
        return jnp.concatenate(parts, axis=0)

    p1_ref[0, 0] = to_patches(b1)
    p2_ref[0, 0] = to_patches(b2)
    p3_ref[0, 0] = to_patches(b3)


def _conv_unfold(x, g_w, g_b, theta_w, theta_b, phi_w, phi_b):
    B = x.shape[0]
    xp = jnp.pad(x, ((0, 0), (0, 0), (1, 3), (1, 1)))
    xp = xp.reshape(B, 96, 57, 4, 226)
    gw = g_w.transpose(2, 3, 0, 1).reshape(9, CI, 96)
    tw = theta_w.reshape(CI, 96)
    pw = phi_w.reshape(CI, 96)
    gb = g_b.reshape(CI, 1)
    tb = theta_b.reshape(CI, 1)
    pb = phi_b.reshape(CI, 1)
    sel = np.zeros((224, 7 * 128), np.float32)
    for kx in range(KSIZE):
        for j in range(PH):
            sel[STRIDE * j + kx, kx * 128 + j] = 1.0
    sel = jnp.asarray(sel)

    def xspec(off):
        return pl.BlockSpec((1, 96, 1, 4, 226),
                            lambda b, i: (b, 0, i + off, 0, 0))
    wspec = lambda s: pl.BlockSpec(s, lambda b, i: tuple(0 for _ in s))
    pspec = pl.BlockSpec((1, 1, D, PH), lambda b, i: (b, i, 0, 0))
    pshape = jax.ShapeDtypeStruct((B, PH, D, PH), jnp.float32)

    slabs = pl.pallas_call(
        _conv_unfold_kernel,
        grid=(B, PH),
        in_specs=[xspec(0), xspec(1), xspec(2),
                  wspec((9, CI, 96)), wspec((CI, 1)),
                  wspec((CI, 96)), wspec((CI, 1)),
                  wspec((CI, 96)), wspec((CI, 1)),
                  wspec((224, 7 * 128))],
        out_specs=[pspec, pspec, pspec],
        out_shape=[pshape, pshape, pshape],
    )(xp, xp, xp, gw, gb, tw, tb, pw, pb, sel)
    return [s.transpose(0, 2, 1, 3).reshape(B, D, L) for s in slabs]


def _attn_kernel(p1_ref, p2_ref, p3_ref, out_ref):
    p1t = p1_ref[0]
    p2b = p2_ref[0]
    p3b = p3_ref[0]

    score = jax.lax.dot_general(
        p1t, p2b, (((0,), (0,)), ((), ())),
        preferred_element_type=jnp.float32)

    mx = jnp.maximum(jnp.max(score, axis=1, keepdims=True) * SCALE, 0.0)

    s = jax.lax.bitcast_convert_type(score, jnp.int32)
    key = jax.lax.bitcast_convert_type(
        s ^ ((s >> 31) | jnp.int32(-2147483648)), jnp.uint32)

    def step(i, acc):
        bit = jax.lax.shift_left(jnp.uint32(1),
                                 jnp.uint32(31) - i.astype(jnp.uint32))
        cand = acc | bit
        cnt = jnp.sum((key >= cand).astype(jnp.int32), axis=1, keepdims=True)
        return jnp.where(cnt >= TOPM, cand, acc)

    acc = jax.lax.fori_loop(0, 32, step,
                            jnp.zeros((p1t.shape[1], 1), jnp.uint32))

    sel = key >= acc
    num = jnp.where(sel, jnp.exp(score * SCALE - mx), jnp.exp(-mx))
    z = jnp.sum(num, axis=1, keepdims=True)

    agg = jax.lax.dot_general(
        p3b, num, (((1,), (1,)), ((), ())),
        preferred_element_type=jnp.float32)
    out_ref[0] = agg * (1.0 / z).reshape(1, -1)


def _fused_attention(p1t, p2t, p3t):
    B = p1t.shape[0]
    grid = (B, pl.cdiv(L, ROWT))
    return pl.pallas_call(
        _attn_kernel,
        grid=grid,
        in_specs=[
            pl.BlockSpec((1, D, ROWT), lambda b, t: (b, 0, t)),
            pl.BlockSpec((1, D, L), lambda b, t: (b, 0, 0)),
            pl.BlockSpec((1, D, L), lambda b, t: (b, 0, 0)),
        ],
        out_specs=pl.BlockSpec((1, D, ROWT), lambda b, t: (b, 0, t)),
        out_shape=jax.ShapeDtypeStruct((B, D, L), jnp.float32),
    )(p1t, p2t, p3t)


def _fold_kernel(acur_ref, aprev_ref, vsel_ref, cntc_ref, mw_ref, mb_ref,
                 rw_ref, rb_ref, out_ref):
    g = pl.program_id(1)
    acur = acur_ref[0, 0]
    aprev = aprev_ref[0, 0]
    wc = jnp.where(g <= PH - 1, 1.0, 0.0).astype(jnp.float32)
    wp = jnp.where(g >= 1, 1.0, 0.0).astype(jnp.float32)

    def dy_row(src, dy):
        groups = [jnp.pad(src[(dy * KSIZE + kx) * CI:(dy * KSIZE + kx + 1) * CI, :],
                          ((0, 0), (0, 128 - PH))) for kx in range(KSIZE)]
        packed = jnp.concatenate(groups, axis=1)
        return jax.lax.dot_general(packed, vsel_ref[...],
                                   (((1,), (0,)), ((), ())),
                                   preferred_element_type=jnp.float32)

    rows = []
    for t in range(4):
        r = dy_row(acur, t) * wc
        if t < 3:
            r = r + dy_row(aprev, t + 4) * wp
        rows.append(r)
    folded = jnp.stack(rows, axis=1)

    gmin = jnp.minimum(g, PH - 1)
    c_low = (gmin - jnp.maximum(g - 1, 0) + 1).astype(jnp.float32)
    c_3 = (gmin - g + 1).astype(jnp.float32)
    cntr = jnp.stack([c_low, c_low, c_low, c_3]).reshape(1, 4, 1)

    denom = (cntr * cntc_ref[...].reshape(1, 1, 224)
             * mw_ref[...].reshape(CI, 1, 1)
             + mb_ref[...].reshape(CI, 1, 1) + 1e-08)
    normed = folded / denom

    out = jax.lax.dot_general(rw_ref[...], normed, (((1,), (0,)), ((), ())),
                              preferred_element_type=jnp.float32)
    out_ref[0, :, 0] = out + rb_ref[...].reshape(96, 1, 1)


def _fold_mask_restore(aggt, match_w, match_b, restore_w, restore_b):
    B = aggt.shape[0]
    a4 = aggt.reshape(B, D, PH, PH).transpose(0, 2, 1, 3)
    cov = np.zeros(224, np.float32)
    for i in range(PH):
        cov[i * STRIDE:i * STRIDE + KSIZE] += 1.0
    cntc = jnp.asarray(cov).reshape(1, 224)
    mw = match_w[:, :, 0, 0].sum(axis=1).reshape(CI, 1)
    mb = match_b.reshape(CI, 1)
    rw = restore_w.reshape(96, CI)
    rb = restore_b.reshape(96, 1)
    vsel = np.zeros((7 * 128, 224), np.float32)
    for kx in range(KSIZE):
        for j in range(PH):
            vsel[kx * 128 + j, STRIDE * j + kx] = 1.0
    vsel = jnp.asarray(vsel)

    wspec = lambda s: pl.BlockSpec(s, lambda b, g: tuple(0 for _ in s))
    out = pl.pallas_call(
        _fold_kernel,
        grid=(B, 56),
        in_specs=[
            pl.BlockSpec((1, 1, D, PH),
                         lambda b, g: (b, jnp.minimum(g, PH - 1), 0, 0)),
            pl.BlockSpec((1, 1, D, PH),
                         lambda b, g: (b, jnp.maximum(g - 1, 0), 0, 0)),
            wspec((7 * 128, 224)),
            wspec((1, 224)), wspec((CI, 1)), wspec((CI, 1)),
            wspec((96, CI)), wspec((96, 1)),
        ],
        out_specs=pl.BlockSpec((1, 96, 1, 4, 224),
                               lambda b, g: (b, 0, g, 0, 0)),
        out_shape=jax.ShapeDtypeStruct((B, 96, 56, 4, 224), jnp.float32),
    )(a4, a4, vsel, cntc, mw, mb, rw, rb)
    return out.reshape(B, 96, 224, 224)


@jax.jit
def kernel(x, g_w, g_b, theta_w, theta_b, phi_w, phi_b, match_w, match_b,
           restore_w, restore_b):
    p1t, p2t, p3t = _conv_unfold(x, g_w, g_b, theta_w, theta_b,
                                 phi_w, phi_b)
    aggt = _fused_attention(p1t, p2t, p3t)
    return _fold_mask_restore(aggt, match_w, match_b, restore_w, restore_b)
